# baseline (speedup 1.0000x reference)
import jax
import jax.numpy as jnp
from jax.experimental import pallas as pl
from jax.experimental.pallas import tpu as pltpu

AUG_T = 1024
B, L, D = 16, 4096, 128
RPB = 2
CH = 512


def _body(lens_ref, x_ref, o_ref):
    i = pl.program_id(0)

    for r in range(RPB):
        slen = lens_ref[i * RPB + r]
        is_long = slen > AUG_T

        for c in range(L // CH):
            base = c * CH
            below = is_long & (base + CH <= slen)
            straddle = is_long & (base < slen) & (slen < base + CH)

            @pl.when(below)
            def _full(r=r, base=base):
                pos = jax.lax.broadcasted_iota(jnp.int32, (CH, 1), 0) + base
                gain = jnp.where(pos % 10 == 0, 0.0, 1.0)
                o_ref[r, pl.ds(base, CH), :] = x_ref[r, pl.ds(base, CH), :] * gain

            @pl.when(straddle)
            def _edge(r=r, base=base):
                pos = jax.lax.broadcasted_iota(jnp.int32, (CH, 1), 0) + base
                gain = jnp.where((pos % 10 == 0) & (pos < slen), 0.0, 1.0)
                o_ref[r, pl.ds(base, CH), :] = x_ref[r, pl.ds(base, CH), :] * gain

            @pl.when(jnp.logical_not(below | straddle))
            def _copy(r=r, base=base):
                o_ref[r, pl.ds(base, CH), :] = x_ref[r, pl.ds(base, CH), :]


def kernel(sequences, seq_lens):
    out = pl.pallas_call(
        _body,
        grid=(B // RPB,),
        in_specs=[
            pl.BlockSpec(memory_space=pltpu.SMEM),
            pl.BlockSpec((RPB, L, D), lambda i: (i, 0, 0)),
        ],
        out_specs=pl.BlockSpec((RPB, L, D), lambda i: (i, 0, 0)),
        out_shape=jax.ShapeDtypeStruct((B, L, D), jnp.float32),
        compiler_params=pltpu.CompilerParams(
            dimension_semantics=("arbitrary",),
        ),
    )(seq_lens, sequences)
    return out, seq_lens


# speedup vs baseline: 1.0487x; 1.0487x over previous
import jax
import jax.numpy as jnp
from jax.experimental import pallas as pl
from jax.experimental.pallas import tpu as pltpu

AUG_T = 1024
B, L, D = 16, 4096, 128
RPB = 2
CH = 512


def _body(lens_ref, x_ref, o_ref, tpl_ref):
    i = pl.program_id(0)

    @pl.when(i == 0)
    def _build_template():
        pos = jax.lax.broadcasted_iota(jnp.int32, (L, D), 0)
        tpl_ref[...] = jnp.where(pos % 10 == 0, 0.0, 1.0)

    for r in range(RPB):
        slen = lens_ref[i * RPB + r]
        is_long = slen > AUG_T

        for c in range(L // CH):
            base = c * CH
            below = is_long & (base + CH <= slen)
            straddle = is_long & (base < slen) & (slen < base + CH)

            @pl.when(below)
            def _full(r=r, base=base):
                o_ref[r, pl.ds(base, CH), :] = (
                    x_ref[r, pl.ds(base, CH), :] * tpl_ref[pl.ds(base, CH), :]
                )

            @pl.when(straddle)
            def _edge(r=r, base=base):
                pos = jax.lax.broadcasted_iota(jnp.int32, (CH, 1), 0) + base
                gain = jnp.where(
                    pos < slen, tpl_ref[pl.ds(base, CH), :], 1.0
                )
                o_ref[r, pl.ds(base, CH), :] = x_ref[r, pl.ds(base, CH), :] * gain

            @pl.when(jnp.logical_not(below | straddle))
            def _copy(r=r, base=base):
                o_ref[r, pl.ds(base, CH), :] = x_ref[r, pl.ds(base, CH), :]


def kernel(sequences, seq_lens):
    out = pl.pallas_call(
        _body,
        grid=(B // RPB,),
        in_specs=[
            pl.BlockSpec(memory_space=pltpu.SMEM),
            pl.BlockSpec((RPB, L, D), lambda i: (i, 0, 0)),
        ],
        out_specs=pl.BlockSpec((RPB, L, D), lambda i: (i, 0, 0)),
        out_shape=jax.ShapeDtypeStruct((B, L, D), jnp.float32),
        scratch_shapes=[pltpu.VMEM((L, D), jnp.float32)],
        compiler_params=pltpu.CompilerParams(
            dimension_semantics=("arbitrary",),
        ),
    )(seq_lens, sequences)
    return out, seq_lens


# speedup vs baseline: 1.0733x; 1.0235x over previous
import jax
import jax.numpy as jnp
from jax.experimental import pallas as pl
from jax.experimental.pallas import tpu as pltpu

AUG_T = 1024
B, L, D = 16, 4096, 128
RPB = 2
CH = 512


def _body(lens_ref, x_ref, o_ref):
    i = pl.program_id(0)

    for r in range(RPB):
        slen = lens_ref[i * RPB + r]
        is_long = slen > AUG_T

        for c in range(L // CH):
            base = c * CH
            masked = is_long & (base < slen)

            @pl.when(masked)
            def _mask(r=r, base=base):
                o_ref[r, pl.ds(base, CH), :] = x_ref[r, pl.ds(base, CH), :]
                first = -(-base // 10) * 10
                for p in range(first, base + CH, 10):
                    o_ref[r, pl.ds(p, 1), :] = jnp.where(
                        p < slen, 0.0, x_ref[r, pl.ds(p, 1), :]
                    )

            @pl.when(jnp.logical_not(masked))
            def _copy(r=r, base=base):
                o_ref[r, pl.ds(base, CH), :] = x_ref[r, pl.ds(base, CH), :]


def kernel(sequences, seq_lens):
    out = pl.pallas_call(
        _body,
        grid=(B // RPB,),
        in_specs=[
            pl.BlockSpec(memory_space=pltpu.SMEM),
            pl.BlockSpec((RPB, L, D), lambda i: (i, 0, 0)),
        ],
        out_specs=pl.BlockSpec((RPB, L, D), lambda i: (i, 0, 0)),
        out_shape=jax.ShapeDtypeStruct((B, L, D), jnp.float32),
        compiler_params=pltpu.CompilerParams(
            dimension_semantics=("arbitrary",),
        ),
    )(seq_lens, sequences)
    return out, seq_lens


# speedup vs baseline: 1.1499x; 1.0713x over previous
import jax
import jax.numpy as jnp
from jax.experimental import pallas as pl
from jax.experimental.pallas import tpu as pltpu

AUG_T = 1024
B, L, D = 16, 4096, 128
RPB = 4
CH = 512


def _body(lens_ref, x_ref, o_ref):
    i = pl.program_id(0)

    for r in range(RPB):
        slen = lens_ref[i * RPB + r]
        is_long = slen > AUG_T

        for c in range(L // CH):
            base = c * CH
            masked = is_long & (base < slen)

            @pl.when(masked)
            def _mask(r=r, base=base):
                o_ref[r, pl.ds(base, CH), :] = x_ref[r, pl.ds(base, CH), :]
                first = -(-base // 10) * 10
                for p in range(first, base + CH, 10):
                    o_ref[r, pl.ds(p, 1), :] = jnp.where(
                        p < slen, 0.0, x_ref[r, pl.ds(p, 1), :]
                    )

            @pl.when(jnp.logical_not(masked))
            def _copy(r=r, base=base):
                o_ref[r, pl.ds(base, CH), :] = x_ref[r, pl.ds(base, CH), :]


def kernel(sequences, seq_lens):
    out = pl.pallas_call(
        _body,
        grid=(B // RPB,),
        in_specs=[
            pl.BlockSpec(memory_space=pltpu.SMEM),
            pl.BlockSpec((RPB, L, D), lambda i: (i, 0, 0)),
        ],
        out_specs=pl.BlockSpec((RPB, L, D), lambda i: (i, 0, 0)),
        out_shape=jax.ShapeDtypeStruct((B, L, D), jnp.float32),
        compiler_params=pltpu.CompilerParams(
            dimension_semantics=("arbitrary",),
        ),
    )(seq_lens, sequences)
    return out, seq_lens
